# baseline (speedup 1.0000x reference)
import functools

import jax
import jax.numpy as jnp
from jax import lax
from jax.experimental import pallas as pl
from jax.experimental.pallas import tpu as pltpu
from jax.experimental.pallas import tpu_sc as plsc

B = 16384
D = 32
NC = 2
NS = 16
NW = NC * NS
BPW = B // NW
L = 16
GROUPS = BPW // L

_LN2 = 0.6931471805599453
_C1 = _LN2 / (1 << 23)
_C2 = 126.94269504 * _LN2


def _softplus16(x):
    w = jnp.exp(-jnp.abs(x))
    z = 1.0 + w
    zb = plsc.bitcast(z, jnp.int32)
    y0 = zb.astype(jnp.float32) * _C1 - _C2
    t = z * jnp.exp(-y0) - 1.0
    corr = t * (1.0 + t * (-0.5 + t * (1.0 / 3.0 + t * -0.25)))
    return jnp.maximum(x, 0.0) + y0 + corr


def _body(user_h, item_h, theta_h, a_h, b_h, out_h,
          uidx_v, iidx_v, th_v, a_v, b_v, o_v, sem):
    wid = lax.axis_index("s") * NC + lax.axis_index("c")
    base = wid * BPW
    pltpu.sync_copy(user_h.at[pl.ds(base, BPW)], uidx_v)
    pltpu.sync_copy(item_h.at[pl.ds(base, BPW)], iidx_v)
    cp_th = pltpu.async_copy(theta_h.at[uidx_v], th_v, sem)
    cp_a = pltpu.async_copy(a_h.at[iidx_v], a_v, sem)
    cp_b = pltpu.async_copy(b_h.at[iidx_v], b_v, sem)
    cp_th.wait()
    cp_a.wait()
    cp_b.wait()

    def group(g, _):
        rows = g * L + lax.iota(jnp.int32, L)
        acc = jnp.zeros((L,), jnp.float32)
        for d in range(D):
            col = jnp.full((L,), d, jnp.int32)
            th = plsc.load_gather(th_v, [rows, col])
            av = plsc.load_gather(a_v, [rows, col])
            acc = acc + _softplus16(av) * th
        bb = b_v[pl.ds(g * L, L)]
        s = acc - bb
        o_v[pl.ds(g * L, L)] = 1.0 / (1.0 + jnp.exp(-s))
        return 0

    lax.fori_loop(0, GROUPS, group, 0)
    pltpu.sync_copy(o_v, out_h.at[pl.ds(base, BPW)])


@jax.jit
def _mirt_sc(user, item, theta_w, a_w, b_flat):
    mesh = plsc.VectorSubcoreMesh(
        core_axis_name="c", subcore_axis_name="s", num_cores=NC, num_subcores=NS
    )
    f = pl.kernel(
        _body,
        out_type=jax.ShapeDtypeStruct((B,), jnp.float32),
        mesh=mesh,
        scratch_types=[
            pltpu.VMEM((BPW,), jnp.int32),
            pltpu.VMEM((BPW,), jnp.int32),
            pltpu.VMEM((BPW, D), jnp.float32),
            pltpu.VMEM((BPW, D), jnp.float32),
            pltpu.VMEM((BPW,), jnp.float32),
            pltpu.VMEM((BPW,), jnp.float32),
            pltpu.SemaphoreType.DMA,
        ],
        compiler_params=pltpu.CompilerParams(
            needs_layout_passes=False, use_tc_tiling_on_sc=False
        ),
    )
    return f(user, item, theta_w, a_w, b_flat)


def kernel(user, item, theta_w, a_w, b_w):
    return _mirt_sc(user, item, theta_w, a_w, jnp.reshape(b_w, (-1,)))
